# baseline (speedup 1.0000x reference)
import jax
import jax.numpy as jnp
from jax.experimental import pallas as pl
from jax.experimental.pallas import tpu as pltpu

HIDDEN = 2048
INTER = 512
INTER_SHARED = 2048
NUM_EXPERTS = 64
TOP_K = 8
NEG_INF = -1e30

J_SHARED = 16
CHUNK_SHARED = INTER_SHARED // J_SHARED


def _routing_from_logits(logits):
    iota = jax.lax.broadcasted_iota(jnp.int32, logits.shape, 1)
    vals = logits
    sel = jnp.zeros(logits.shape, jnp.bool_)
    for _ in range(TOP_K):
        m = jnp.max(vals, axis=-1, keepdims=True)
        cand = jnp.where(vals == m, iota, NUM_EXPERTS)
        idx = jnp.min(cand, axis=-1, keepdims=True)
        pick = iota == idx
        sel = jnp.logical_or(sel, pick)
        vals = jnp.where(pick, NEG_INF, vals)
    mtop = jnp.max(jnp.where(sel, logits, NEG_INF), axis=-1, keepdims=True)
    ex = jnp.where(sel, jnp.exp(logits - mtop), 0.0)
    return ex / jnp.sum(ex, axis=-1, keepdims=True)


def _fused_kernel(x_ref, rw_ref, sgw_ref, gw_ref, iw_ref, sow_ref,
                  gu_ref, ow_ref, out_ref, sacc_ref, rt_ref):
    e = pl.program_id(0)
    x = x_ref[:]

    def _shared_chunk():
        g = jax.nn.silu(jnp.dot(x, gw_ref[:],
                                preferred_element_type=jnp.float32))
        i = jnp.dot(x, iw_ref[:], preferred_element_type=jnp.float32)
        return jnp.dot(g * i, sow_ref[:], preferred_element_type=jnp.float32)

    @pl.when(e == 0)
    def _init():
        logits = jnp.dot(x, rw_ref[:], preferred_element_type=jnp.float32)
        rt_ref[:] = _routing_from_logits(logits)
        sacc_ref[:] = _shared_chunk()

    @pl.when(jnp.logical_and(e > 0, e < J_SHARED))
    def _shared_acc():
        sacc_ref[:] += _shared_chunk()

    iota = jax.lax.broadcasted_iota(jnp.int32, rt_ref.shape, 1)
    contrib = None
    for sub in range(2):
        xw = jnp.dot(x, gu_ref[sub], preferred_element_type=jnp.float32)
        gate = xw[:, :INTER]
        up = xw[:, INTER:]
        h = up * jax.nn.silu(gate)
        w = jnp.sum(jnp.where(iota == 2 * e + sub, rt_ref[:], 0.0),
                    axis=-1, keepdims=True)
        c = jnp.dot(h * w, ow_ref[sub], preferred_element_type=jnp.float32)
        contrib = c if contrib is None else contrib + c

    @pl.when(e == 0)
    def _out_init():
        out_ref[:] = contrib

    @pl.when(e > 0)
    def _out_acc():
        out_ref[:] += contrib

    @pl.when(e == NUM_EXPERTS // 2 - 1)
    def _fin():
        sg = jax.nn.sigmoid(
            jnp.dot(x, sgw_ref[:], preferred_element_type=jnp.float32))
        out_ref[:] += sg * sacc_ref[:]


def _moe(x, router_w, expert_gate_up, expert_out_w, shared_gate_w,
         shared_inter_w, shared_out_w, shared_expert_gate_w, interpret=False):
    T = x.shape[0]
    jcap = J_SHARED - 1
    out = pl.pallas_call(
        _fused_kernel,
        grid=(NUM_EXPERTS // 2,),
        in_specs=[
            pl.BlockSpec((T, HIDDEN), lambda e: (0, 0)),
            pl.BlockSpec((HIDDEN, NUM_EXPERTS), lambda e: (0, 0)),
            pl.BlockSpec((HIDDEN, 1), lambda e: (0, 0)),
            pl.BlockSpec((HIDDEN, CHUNK_SHARED),
                         lambda e: (0, jnp.minimum(e, jcap))),
            pl.BlockSpec((HIDDEN, CHUNK_SHARED),
                         lambda e: (0, jnp.minimum(e, jcap))),
            pl.BlockSpec((CHUNK_SHARED, HIDDEN),
                         lambda e: (jnp.minimum(e, jcap), 0)),
            pl.BlockSpec((2, HIDDEN, 2 * INTER), lambda e: (e, 0, 0)),
            pl.BlockSpec((2, INTER, HIDDEN), lambda e: (e, 0, 0)),
        ],
        out_specs=pl.BlockSpec((T, HIDDEN), lambda e: (0, 0)),
        out_shape=jax.ShapeDtypeStruct((T, HIDDEN), jnp.float32),
        scratch_shapes=[
            pltpu.VMEM((T, HIDDEN), jnp.float32),
            pltpu.VMEM((T, NUM_EXPERTS), jnp.float32),
        ],
        compiler_params=pltpu.CompilerParams(
            dimension_semantics=("arbitrary",)),
        interpret=interpret,
    )(x, router_w, shared_expert_gate_w, shared_gate_w, shared_inter_w,
      shared_out_w, expert_gate_up, expert_out_w)
    return out


def kernel(hidden_states, router_w, expert_gate_up, expert_out_w,
           shared_gate_w, shared_inter_w, shared_out_w, shared_expert_gate_w):
    b, s, h = hidden_states.shape
    x = hidden_states.reshape(-1, h)
    out = _moe(x, router_w, expert_gate_up, expert_out_w, shared_gate_w,
               shared_inter_w, shared_out_w, shared_expert_gate_w)
    return out.reshape(b, s, h)
